# baseline (speedup 1.0000x reference)
import functools

import jax
import jax.numpy as jnp
from jax import lax
from jax.experimental import pallas as pl
from jax.experimental.pallas import tpu as pltpu
from jax.experimental.pallas import tpu_sc as plsc

B = 8
F = 128
H = 50
W = 50
SEGS = ((0, 13), (13, 13), (26, 12), (38, 12))


def _expand_segment(traw_v, tab_v, seg_off, seg_len):
    def body(p, carry):
        for kk in range(F // 16):
            v = traw_v[p, pl.ds(kk * 16, 16)]
            for r in range(B):
                tab_v[p, r, pl.ds(kk * 16, 16)] = v
        return carry

    lax.fori_loop(seg_off, seg_off + seg_len, body, 0)


def _half_program(tbl_hbm, out_hbm, traw_v, tab_v, sem, sem_out, s, half):
    pltpu.async_copy(tbl_hbm, traw_v, sem).wait()

    def dst(k, seg_off, seg_len):
        if half == 0:
            return out_hbm.at[k, pl.ds(seg_off, seg_len), 0]
        return out_hbm.at[pl.ds(seg_off, seg_len), k, 1]

    copies = []
    for seg_off, seg_len in SEGS:
        _expand_segment(traw_v, tab_v, seg_off, seg_len)
        src = tab_v.at[pl.ds(seg_off, seg_len)]
        for i in range(3):
            copies.append(
                pltpu.async_copy(src, dst(s + 16 * i, seg_off, seg_len), sem_out)
            )

        @pl.when(s < 2)
        def _():
            pltpu.async_copy(src, dst(48 + s, seg_off, seg_len), sem_out).wait()

    for c in copies:
        c.wait()


def _pos_body(row_hbm, col_hbm, out_hbm, traw_v, tab_v, sem, sem_out):
    h = lax.axis_index("c")
    s = lax.axis_index("s")

    @pl.when(h == 0)
    def _():
        _half_program(row_hbm, out_hbm, traw_v, tab_v, sem, sem_out, s, 0)

    @pl.when(h == 1)
    def _():
        _half_program(col_hbm, out_hbm, traw_v, tab_v, sem, sem_out, s, 1)


@jax.jit
def _build_pos(row_embed, col_embed):
    mesh = plsc.VectorSubcoreMesh(core_axis_name="c", subcore_axis_name="s")
    k = functools.partial(
        pl.kernel,
        mesh=mesh,
        compiler_params=pltpu.CompilerParams(needs_layout_passes=False),
        out_type=jax.ShapeDtypeStruct((H, W, 2, B, F), jnp.float32),
        scratch_types=[
            pltpu.VMEM((H, F), jnp.float32),
            pltpu.VMEM((H, B, F), jnp.float32),
            pltpu.SemaphoreType.DMA,
            pltpu.SemaphoreType.DMA,
        ],
    )(_pos_body)
    return k(row_embed, col_embed)


def kernel(uv_feat, row_embed, col_embed):
    tiles = _build_pos(row_embed, col_embed)
    out = tiles.transpose(3, 2, 4, 0, 1)
    return out.reshape(B, 2 * F, H, W)
